# baseline (speedup 1.0000x reference)
import jax
import jax.numpy as jnp
import numpy as np
from jax import lax
from jax.experimental import pallas as pl
from jax.experimental.pallas import tpu as pltpu
from jax.experimental.pallas import tpu_sc as plsc

_EPS = float(np.finfo(np.float64).eps)
_CLAMP = float(0.5 + 0.5 * np.tanh(np.float32(np.log(_EPS)) / 2.0))

_N = 4194304
_NTILES = 32
_PER_TILE = _N // _NTILES
_CH = 16384
_NCH = _PER_TILE // _CH

_mesh = plsc.VectorSubcoreMesh(
    core_axis_name="c", subcore_axis_name="s", num_cores=2, num_subcores=16)


def _bcast(ref, i):
    return plsc.load_gather(ref, [jnp.full((16,), i, jnp.int32)])


def _classify(bb_ref, b3v, b7v, b11v, x):
    c1 = b7v <= x
    pos = jnp.where(c1, 8, 0)
    bv2 = jnp.where(c1, b11v, b3v)
    pos = jnp.where(bv2 <= x, pos + 4, pos)
    for step in (2, 1):
        probe = pos + (step - 1)
        bv = plsc.load_gather(bb_ref, [probe])
        pos = jnp.where(bv <= x, pos + step, pos)
    return pos


def _classify16(bbt_ref, b3v, b7v, b11v, lane, x):
    c1 = b7v <= x
    pos = jnp.where(c1, 128, 0)
    bv2 = jnp.where(c1, b11v, b3v)
    pos = jnp.where(bv2 <= x, pos + 64, pos)
    for stp in (32, 16):
        probe = jnp.bitwise_or(pos + (stp - 16), lane)
        bv = plsc.load_gather(bbt_ref, [probe])
        pos = jnp.where(bv <= x, pos + stp, pos)
    return pos


def _hist_body(x_hbm, bb_hbm, bbt_hbm, hist_hbm, bb_v, bbt, xb0, xb1, stab,
               ctab, out32, sem0, sem1):
    wid = lax.axis_index("s") * 2 + lax.axis_index("c")
    base = wid * _PER_TILE
    pltpu.sync_copy(bb_hbm, bb_v)
    pltpu.sync_copy(bbt_hbm, bbt)
    zero16 = jnp.zeros((16,), jnp.float32)
    for j in range(16):
        stab[pl.ds(j * 16, 16)] = zero16
        ctab[pl.ds(j * 16, 16)] = zero16
    lane = lax.iota(jnp.int32, 16)
    lane16 = lane * 16
    ones = jnp.ones((16,), jnp.float32)
    b3v = _bcast(bb_v, 3)
    b7v = _bcast(bb_v, 7)
    b11v = _bcast(bb_v, 11)

    bufs = (xb0, xb1)
    sems = (sem0, sem1)
    copies = [None, None]
    copies[0] = pltpu.async_copy(x_hbm.at[pl.ds(base, _CH)], xb0, sem0)
    for ch in range(_NCH):
        cur = bufs[ch % 2]
        if ch + 1 < _NCH:
            copies[(ch + 1) % 2] = pltpu.async_copy(
                x_hbm.at[pl.ds(base + (ch + 1) * _CH, _CH)],
                bufs[(ch + 1) % 2], sems[(ch + 1) % 2])
        copies[ch % 2].wait()

        @plsc.parallel_loop(0, _CH // 16, step=1, unroll=8)
        def _(v):
            off = pl.multiple_of(v * 16, 16)
            x = cur[pl.ds(off, 16)]
            p = 1.0 / (1.0 + jnp.exp(x * (-1.0)))
            pos = _classify16(bbt, b3v, b7v, b11v, lane, x)
            idx2 = jnp.bitwise_or(pos, lane)
            plsc.addupdate_scatter(stab, [idx2], p)
            plsc.addupdate_scatter(ctab, [idx2], ones)

    svec = jnp.zeros((16,), jnp.float32)
    cvec = jnp.zeros((16,), jnp.float32)
    for l in range(16):
        gidx = lane16 + l
        svec = svec + plsc.load_gather(stab, [gidx])
        cvec = cvec + plsc.load_gather(ctab, [gidx])
    out32[pl.ds(0, 16)] = svec
    out32[pl.ds(16, 16)] = cvec
    pltpu.sync_copy(out32, hist_hbm.at[pl.ds(wid * 32, 32)])


def _apply_body(x_hbm, bb_hbm, bbt_hbm, hist_hbm, y_hbm, bb_v, bbt, htab,
                tau, taut, xb0, xb1, ob0, ob1, sem0, sem1, osem0, osem1):
    wid = lax.axis_index("s") * 2 + lax.axis_index("c")
    base = wid * _PER_TILE
    pltpu.sync_copy(bb_hbm, bb_v)
    pltpu.sync_copy(bbt_hbm, bbt)
    pltpu.sync_copy(hist_hbm, htab)

    S = jnp.zeros((16,), jnp.float32)
    C = jnp.zeros((16,), jnp.float32)
    for w in range(_NTILES):
        S = S + htab[pl.ds(w * 32, 16)]
        C = C + htab[pl.ds(w * 32 + 16, 16)]
    gtot = jnp.sum(S)
    mean_w = gtot * (1.0 / float(_N))
    mean_v = jnp.full((16,), mean_w, jnp.float32)
    pp = jnp.where(C > 0.0, S / jnp.maximum(C, 1.0), mean_v)
    num = pp
    den = 1.0 - pp
    a = jnp.maximum(num, _EPS)
    b = jnp.maximum(den, _EPS)
    t = a / (a + b)
    t = jnp.where((num == 0.0) | (den == 0.0), _CLAMP, t)
    tau[pl.ds(0, 16)] = t
    lane = lax.iota(jnp.int32, 16)
    lane16 = lane * 16
    for l in range(16):
        plsc.store_scatter(taut, [lane16 + l], t)
    b3v = _bcast(bb_v, 3)
    b7v = _bcast(bb_v, 7)
    b11v = _bcast(bb_v, 11)

    xbufs = (xb0, xb1)
    obufs = (ob0, ob1)
    sems = (sem0, sem1)
    osems = (osem0, osem1)
    icopies = [None, None]
    ocopies = [None, None]
    icopies[0] = pltpu.async_copy(x_hbm.at[pl.ds(base, _CH)], xb0, sem0)
    for ch in range(_NCH):
        cur = xbufs[ch % 2]
        ob = obufs[ch % 2]
        if ch + 1 < _NCH:
            icopies[(ch + 1) % 2] = pltpu.async_copy(
                x_hbm.at[pl.ds(base + (ch + 1) * _CH, _CH)],
                xbufs[(ch + 1) % 2], sems[(ch + 1) % 2])
        icopies[ch % 2].wait()
        if ch >= 2:
            ocopies[ch % 2].wait()

        @plsc.parallel_loop(0, _CH // 16, step=1, unroll=8)
        def _(v):
            off = pl.multiple_of(v * 16, 16)
            x = cur[pl.ds(off, 16)]
            pos = _classify16(bbt, b3v, b7v, b11v, lane, x)
            idx2 = jnp.bitwise_or(pos, lane)
            ob[pl.ds(off, 16)] = plsc.load_gather(taut, [idx2])

        ocopies[ch % 2] = pltpu.async_copy(
            ob, y_hbm.at[pl.ds(base + ch * _CH, _CH)], osems[ch % 2])
    ocopies[(_NCH - 2) % 2].wait()
    ocopies[(_NCH - 1) % 2].wait()


@jax.jit
def kernel(logits, bin_boundaries):
    bb16 = jnp.concatenate(
        [bin_boundaries, jnp.full((2,), jnp.inf, jnp.float32)])
    bbt16 = jnp.repeat(bb16, 16)

    hist = pl.kernel(
        _hist_body,
        out_type=jax.ShapeDtypeStruct((_NTILES * 32,), jnp.float32),
        mesh=_mesh,
        compiler_params=pltpu.CompilerParams(needs_layout_passes=False),
        scratch_types=[
            pltpu.VMEM((16,), jnp.float32),
            pltpu.VMEM((256,), jnp.float32),
            pltpu.VMEM((_CH,), jnp.float32),
            pltpu.VMEM((_CH,), jnp.float32),
            pltpu.VMEM((256,), jnp.float32),
            pltpu.VMEM((256,), jnp.float32),
            pltpu.VMEM((32,), jnp.float32),
            pltpu.SemaphoreType.DMA,
            pltpu.SemaphoreType.DMA,
        ],
    )(logits, bb16, bbt16)

    out = pl.kernel(
        _apply_body,
        out_type=jax.ShapeDtypeStruct((_N,), jnp.float32),
        mesh=_mesh,
        compiler_params=pltpu.CompilerParams(needs_layout_passes=False),
        scratch_types=[
            pltpu.VMEM((16,), jnp.float32),
            pltpu.VMEM((256,), jnp.float32),
            pltpu.VMEM((_NTILES * 32,), jnp.float32),
            pltpu.VMEM((16,), jnp.float32),
            pltpu.VMEM((256,), jnp.float32),
            pltpu.VMEM((_CH,), jnp.float32),
            pltpu.VMEM((_CH,), jnp.float32),
            pltpu.VMEM((_CH,), jnp.float32),
            pltpu.VMEM((_CH,), jnp.float32),
            pltpu.SemaphoreType.DMA,
            pltpu.SemaphoreType.DMA,
            pltpu.SemaphoreType.DMA,
            pltpu.SemaphoreType.DMA,
        ],
    )(logits, bb16, bbt16, hist)

    return out


# speedup vs baseline: 1.0566x; 1.0566x over previous
import jax
import jax.numpy as jnp
import numpy as np
from jax import lax
from jax.experimental import pallas as pl
from jax.experimental.pallas import tpu as pltpu
from jax.experimental.pallas import tpu_sc as plsc

_EPS = float(np.finfo(np.float64).eps)
_CLAMP = float(0.5 + 0.5 * np.tanh(np.float32(np.log(_EPS)) / 2.0))

_N = 4194304
_NTILES = 32
_PER_TILE = _N // _NTILES
_CH = 16384
_NCH = _PER_TILE // _CH

_mesh = plsc.VectorSubcoreMesh(
    core_axis_name="c", subcore_axis_name="s", num_cores=2, num_subcores=16)


def _bcast(ref, i):
    return plsc.load_gather(ref, [jnp.full((16,), i, jnp.int32)])


def _classify(bb_ref, b3v, b7v, b11v, x):
    c1 = b7v <= x
    pos = jnp.where(c1, 8, 0)
    bv2 = jnp.where(c1, b11v, b3v)
    pos = jnp.where(bv2 <= x, pos + 4, pos)
    for step in (2, 1):
        probe = pos + (step - 1)
        bv = plsc.load_gather(bb_ref, [probe])
        pos = jnp.where(bv <= x, pos + step, pos)
    return pos


def _classify16(bbt_ref, b3v, b7v, b11v, lane, x):
    c1 = b7v <= x
    pos = jnp.where(c1, 128, 0)
    bv2 = jnp.where(c1, b11v, b3v)
    pos = jnp.where(bv2 <= x, pos + 64, pos)
    for stp in (32, 16):
        probe = jnp.bitwise_or(pos + (stp - 16), lane)
        bv = plsc.load_gather(bbt_ref, [probe])
        pos = jnp.where(bv <= x, pos + stp, pos)
    return pos


def _hist_body(x_hbm, bb_hbm, bbt_hbm, hist_hbm, bb_v, bbt, xb0, xb1, stab,
               ctab, out32, sem0, sem1):
    wid = lax.axis_index("s") * 2 + lax.axis_index("c")
    base = wid * _PER_TILE
    pltpu.sync_copy(bb_hbm, bb_v)
    pltpu.sync_copy(bbt_hbm, bbt)
    zero16 = jnp.zeros((16,), jnp.float32)
    for j in range(16):
        stab[pl.ds(j * 16, 16)] = zero16
        ctab[pl.ds(j * 16, 16)] = zero16
    lane = lax.iota(jnp.int32, 16)
    lane16 = lane * 16
    ones = jnp.ones((16,), jnp.float32)
    b3v = _bcast(bb_v, 3)
    b7v = _bcast(bb_v, 7)
    b11v = _bcast(bb_v, 11)

    bufs = (xb0, xb1)
    sems = (sem0, sem1)
    copies = [None, None]
    copies[0] = pltpu.async_copy(x_hbm.at[pl.ds(base, _CH)], xb0, sem0)
    for ch in range(_NCH):
        cur = bufs[ch % 2]
        if ch + 1 < _NCH:
            copies[(ch + 1) % 2] = pltpu.async_copy(
                x_hbm.at[pl.ds(base + (ch + 1) * _CH, _CH)],
                bufs[(ch + 1) % 2], sems[(ch + 1) % 2])
        copies[ch % 2].wait()

        @plsc.parallel_loop(0, _CH // 16, step=1, unroll=8)
        def _(v):
            off = pl.multiple_of(v * 16, 16)
            x = cur[pl.ds(off, 16)]
            p = 1.0 / (1.0 + jnp.exp(x * (-1.0)))
            pos = _classify16(bbt, b3v, b7v, b11v, lane, x)
            idx2 = jnp.bitwise_or(pos, lane)
            plsc.addupdate_scatter(stab, [idx2], p)
            plsc.addupdate_scatter(ctab, [idx2], ones)

    svec = jnp.zeros((16,), jnp.float32)
    cvec = jnp.zeros((16,), jnp.float32)
    for l in range(16):
        gidx = lane16 + l
        svec = svec + plsc.load_gather(stab, [gidx])
        cvec = cvec + plsc.load_gather(ctab, [gidx])
    out32[pl.ds(0, 16)] = svec
    out32[pl.ds(16, 16)] = cvec
    pltpu.sync_copy(out32, hist_hbm.at[pl.ds(wid * 32, 32)])


def _apply_body(x_hbm, bb_hbm, hist_hbm, y_hbm, bb_v, htab, tau, xb0, xb1,
                ob0, ob1, sem0, sem1, osem0, osem1):
    wid = lax.axis_index("s") * 2 + lax.axis_index("c")
    base = wid * _PER_TILE
    pltpu.sync_copy(bb_hbm, bb_v)
    pltpu.sync_copy(hist_hbm, htab)

    S = jnp.zeros((16,), jnp.float32)
    C = jnp.zeros((16,), jnp.float32)
    for w in range(_NTILES):
        S = S + htab[pl.ds(w * 32, 16)]
        C = C + htab[pl.ds(w * 32 + 16, 16)]
    gtot = jnp.sum(S)
    mean_w = gtot * (1.0 / float(_N))
    mean_v = jnp.full((16,), mean_w, jnp.float32)
    pp = jnp.where(C > 0.0, S / jnp.maximum(C, 1.0), mean_v)
    num = pp
    den = 1.0 - pp
    a = jnp.maximum(num, _EPS)
    b = jnp.maximum(den, _EPS)
    t = a / (a + b)
    t = jnp.where((num == 0.0) | (den == 0.0), _CLAMP, t)
    tau[pl.ds(0, 16)] = t
    b3v = _bcast(bb_v, 3)
    b7v = _bcast(bb_v, 7)
    b11v = _bcast(bb_v, 11)

    xbufs = (xb0, xb1)
    obufs = (ob0, ob1)
    sems = (sem0, sem1)
    osems = (osem0, osem1)
    icopies = [None, None]
    ocopies = [None, None]
    icopies[0] = pltpu.async_copy(x_hbm.at[pl.ds(base, _CH)], xb0, sem0)
    for ch in range(_NCH):
        cur = xbufs[ch % 2]
        ob = obufs[ch % 2]
        if ch + 1 < _NCH:
            icopies[(ch + 1) % 2] = pltpu.async_copy(
                x_hbm.at[pl.ds(base + (ch + 1) * _CH, _CH)],
                xbufs[(ch + 1) % 2], sems[(ch + 1) % 2])
        icopies[ch % 2].wait()
        if ch >= 2:
            ocopies[ch % 2].wait()

        @plsc.parallel_loop(0, _CH // 16, step=1, unroll=8)
        def _(v):
            off = pl.multiple_of(v * 16, 16)
            x = cur[pl.ds(off, 16)]
            pos = _classify(bb_v, b3v, b7v, b11v, x)
            ob[pl.ds(off, 16)] = plsc.load_gather(tau, [pos])

        ocopies[ch % 2] = pltpu.async_copy(
            ob, y_hbm.at[pl.ds(base + ch * _CH, _CH)], osems[ch % 2])
    ocopies[(_NCH - 2) % 2].wait()
    ocopies[(_NCH - 1) % 2].wait()


@jax.jit
def kernel(logits, bin_boundaries):
    bb16 = jnp.concatenate(
        [bin_boundaries, jnp.full((2,), jnp.inf, jnp.float32)])
    bbt16 = jnp.repeat(bb16, 16)

    hist = pl.kernel(
        _hist_body,
        out_type=jax.ShapeDtypeStruct((_NTILES * 32,), jnp.float32),
        mesh=_mesh,
        compiler_params=pltpu.CompilerParams(needs_layout_passes=False),
        scratch_types=[
            pltpu.VMEM((16,), jnp.float32),
            pltpu.VMEM((256,), jnp.float32),
            pltpu.VMEM((_CH,), jnp.float32),
            pltpu.VMEM((_CH,), jnp.float32),
            pltpu.VMEM((256,), jnp.float32),
            pltpu.VMEM((256,), jnp.float32),
            pltpu.VMEM((32,), jnp.float32),
            pltpu.SemaphoreType.DMA,
            pltpu.SemaphoreType.DMA,
        ],
    )(logits, bb16, bbt16)

    out = pl.kernel(
        _apply_body,
        out_type=jax.ShapeDtypeStruct((_N,), jnp.float32),
        mesh=_mesh,
        compiler_params=pltpu.CompilerParams(needs_layout_passes=False),
        scratch_types=[
            pltpu.VMEM((16,), jnp.float32),
            pltpu.VMEM((_NTILES * 32,), jnp.float32),
            pltpu.VMEM((16,), jnp.float32),
            pltpu.VMEM((_CH,), jnp.float32),
            pltpu.VMEM((_CH,), jnp.float32),
            pltpu.VMEM((_CH,), jnp.float32),
            pltpu.VMEM((_CH,), jnp.float32),
            pltpu.SemaphoreType.DMA,
            pltpu.SemaphoreType.DMA,
            pltpu.SemaphoreType.DMA,
            pltpu.SemaphoreType.DMA,
        ],
    )(logits, bb16, hist)

    return out
